# baseline (speedup 1.0000x reference)
import jax
import jax.numpy as jnp
from jax import lax
from jax.experimental import pallas as pl
from jax.experimental.pallas import tpu as pltpu
from jax.experimental.pallas import tpu_sc as plsc

_T = 8192
_D = 768
_NW = 32
_RPW = _T // _NW
_CH = 64
_NCH = _RPW // _CH
_NBUF = 2


def _make_sc_copy():
    mesh = plsc.VectorSubcoreMesh(core_axis_name="c", subcore_axis_name="s")

    def body(wpe_hbm, out_hbm, *scratch):
        bufs = scratch[:_NBUF]
        rsems = scratch[_NBUF:2 * _NBUF]
        wsems = scratch[2 * _NBUF:3 * _NBUF]
        wid = lax.axis_index("s") * 2 + lax.axis_index("c")
        base = wid * _RPW

        def rd(i):
            return pltpu.async_copy(
                wpe_hbm.at[pl.ds(base + i * _CH, _CH)],
                bufs[i % _NBUF], rsems[i % _NBUF])

        def wr(i):
            return pltpu.async_copy(
                bufs[i % _NBUF],
                out_hbm.at[pl.ds(base + i * _CH, _CH)], wsems[i % _NBUF])

        reads = {j: rd(j) for j in range(_NBUF)}
        writes = {}
        for i in range(_NCH):
            reads[i].wait()
            writes[i] = wr(i)
            nxt = i + _NBUF
            if nxt < _NCH:
                writes[i].wait()
                reads[nxt] = rd(nxt)
        for i in range(max(0, _NCH - _NBUF), _NCH):
            writes[i].wait()

    return pl.kernel(
        body,
        out_type=jax.ShapeDtypeStruct((_T, _D), jnp.float32),
        mesh=mesh,
        scratch_types=(
            [pltpu.VMEM((_CH, _D), jnp.float32) for _ in range(_NBUF)]
            + [pltpu.SemaphoreType.DMA for _ in range(2 * _NBUF)]
        ),
    )


_sc_copy = _make_sc_copy()


def kernel(idx, wpe):
    del idx
    return _sc_copy(wpe)


# speedup vs baseline: 1.0144x; 1.0144x over previous
import jax
import jax.numpy as jnp
from jax import lax
from jax.experimental import pallas as pl
from jax.experimental.pallas import tpu as pltpu
from jax.experimental.pallas import tpu_sc as plsc

_T = 8192
_D = 768
_NW = 32
_RPW = _T // _NW
_CH = 16
_NCH = _RPW // _CH
_NBUF = 8


def _make_sc_copy():
    mesh = plsc.VectorSubcoreMesh(core_axis_name="c", subcore_axis_name="s")

    def body(wpe_hbm, out_hbm, *scratch):
        bufs = scratch[:_NBUF]
        rsems = scratch[_NBUF:2 * _NBUF]
        wsems = scratch[2 * _NBUF:3 * _NBUF]
        wid = lax.axis_index("s") * 2 + lax.axis_index("c")
        base = wid * _RPW

        def rd(i):
            return pltpu.async_copy(
                wpe_hbm.at[pl.ds(base + i * _CH, _CH)],
                bufs[i % _NBUF], rsems[i % _NBUF])

        def wr(i):
            return pltpu.async_copy(
                bufs[i % _NBUF],
                out_hbm.at[pl.ds(base + i * _CH, _CH)], wsems[i % _NBUF])

        reads = {j: rd(j) for j in range(_NBUF)}
        writes = {}
        for i in range(_NCH):
            reads[i].wait()
            writes[i] = wr(i)
            nxt = i + _NBUF
            if nxt < _NCH:
                writes[i].wait()
                reads[nxt] = rd(nxt)
        for i in range(max(0, _NCH - _NBUF), _NCH):
            writes[i].wait()

    return pl.kernel(
        body,
        out_type=jax.ShapeDtypeStruct((_T, _D), jnp.float32),
        mesh=mesh,
        scratch_types=(
            [pltpu.VMEM((_CH, _D), jnp.float32) for _ in range(_NBUF)]
            + [pltpu.SemaphoreType.DMA for _ in range(2 * _NBUF)]
        ),
    )


_sc_copy = _make_sc_copy()


def kernel(idx, wpe):
    del idx
    return _sc_copy(wpe)


# speedup vs baseline: 1.0252x; 1.0107x over previous
import jax
import jax.numpy as jnp
from jax import lax
from jax.experimental import pallas as pl
from jax.experimental.pallas import tpu as pltpu
from jax.experimental.pallas import tpu_sc as plsc

_T = 8192
_D = 768
_NW = 32
_RPW = _T // _NW
_CH = 32
_NCH = _RPW // _CH
_NBUF = 4


def _make_sc_copy():
    mesh = plsc.VectorSubcoreMesh(core_axis_name="c", subcore_axis_name="s")

    def body(wpe_hbm, out_hbm, buf, rsems, wsems):
        wid = lax.axis_index("s") * 2 + lax.axis_index("c")
        base = wid * _RPW

        def rd(i, slot):
            return pltpu.make_async_copy(
                wpe_hbm.at[pl.ds(base + i * _CH, _CH)], buf.at[slot],
                rsems.at[slot])

        def wr(i, slot):
            return pltpu.make_async_copy(
                buf.at[slot], out_hbm.at[pl.ds(base + i * _CH, _CH)],
                wsems.at[slot])

        for j in range(_NBUF):
            rd(j, j).start()

        def step(i, carry):
            slot = lax.rem(i, _NBUF)
            rd(i, slot).wait()
            wr(i, slot).start()

            @pl.when(i + _NBUF < _NCH)
            def _():
                wr(i, slot).wait()
                rd(i + _NBUF, slot).start()

            return carry

        lax.fori_loop(0, _NCH, step, 0, unroll=False)

        def drain(i, carry):
            wr(i, lax.rem(i, _NBUF)).wait()
            return carry

        lax.fori_loop(_NCH - _NBUF, _NCH, drain, 0, unroll=False)

    return pl.kernel(
        body,
        out_type=jax.ShapeDtypeStruct((_T, _D), jnp.float32),
        mesh=mesh,
        scratch_types=[
            pltpu.VMEM((_NBUF, _CH, _D), jnp.float32),
            pltpu.SemaphoreType.DMA((_NBUF,)),
            pltpu.SemaphoreType.DMA((_NBUF,)),
        ],
    )


_sc_copy = _make_sc_copy()


def kernel(idx, wpe):
    del idx
    return _sc_copy(wpe)


# speedup vs baseline: 1.0473x; 1.0215x over previous
import jax
import jax.numpy as jnp
from jax import lax
from jax.experimental import pallas as pl
from jax.experimental.pallas import tpu as pltpu
from jax.experimental.pallas import tpu_sc as plsc

_T = 8192
_D = 768
_NW = 32
_RPW = _T // _NW
_CH = 16
_NCH = _RPW // _CH
_NBUF = 8
_LAG = 4


def _make_sc_copy():
    mesh = plsc.VectorSubcoreMesh(core_axis_name="c", subcore_axis_name="s")

    def body(wpe_hbm, out_hbm, buf, rsems, wsems):
        wid = lax.axis_index("s") * 2 + lax.axis_index("c")
        base = wid * _RPW

        def rd(i, slot):
            return pltpu.make_async_copy(
                wpe_hbm.at[pl.ds(base + i * _CH, _CH)], buf.at[slot],
                rsems.at[slot])

        def wr(i, slot):
            return pltpu.make_async_copy(
                buf.at[slot], out_hbm.at[pl.ds(base + i * _CH, _CH)],
                wsems.at[slot])

        for j in range(_NBUF - _LAG):
            rd(j, j).start()

        def step(i, carry):
            @pl.when(i >= _LAG)
            def _():
                wr(i - _LAG, lax.rem(i - _LAG, _NBUF)).wait()

            nxt = i + _NBUF - _LAG

            @pl.when(nxt < _NCH)
            def _():
                rd(nxt, lax.rem(nxt, _NBUF)).start()

            slot = lax.rem(i, _NBUF)
            rd(i, slot).wait()
            wr(i, slot).start()
            return carry

        lax.fori_loop(0, _NCH, step, 0, unroll=False)

        def drain(i, carry):
            wr(i, lax.rem(i, _NBUF)).wait()
            return carry

        lax.fori_loop(max(0, _NCH - _LAG), _NCH, drain, 0, unroll=False)

    return pl.kernel(
        body,
        out_type=jax.ShapeDtypeStruct((_T, _D), jnp.float32),
        mesh=mesh,
        scratch_types=[
            pltpu.VMEM((_NBUF, _CH, _D), jnp.float32),
            pltpu.SemaphoreType.DMA((_NBUF,)),
            pltpu.SemaphoreType.DMA((_NBUF,)),
        ],
    )


_sc_copy = _make_sc_copy()


def kernel(idx, wpe):
    del idx
    return _sc_copy(wpe)
